# baseline (speedup 1.0000x reference)
import functools

import jax
import jax.numpy as jnp
from jax import lax
from jax.experimental import pallas as pl
from jax.experimental.pallas import tpu as pltpu
from jax.experimental.pallas import tpu_sc as plsc

N = 10000
NP = 10112
NE = 320000
F = 128
NG = 64
NOUT = 10

NC, NS = 2, 16
FH = F // NC
CHUNK = 128
NCHUNK = 160
NEP = CHUNK * NCHUNK * NS
E2D = NEP // CHUNK
DCHUNK = E2D // (NC * NS)
RPT = NP // NS
DEGW = 16


def _deg_body(dst_hbm, out_hbm, dst_v, ones_v, z_v, acc_sh):
    cid = lax.axis_index("c")
    sid = lax.axis_index("s")
    wid = cid * NS + sid
    row0 = sid * RPT

    @pl.loop(0, CHUNK)
    def _(r):
        ones_v[r, pl.ds(0, DEGW)] = jnp.ones((16,), jnp.float32)

    @pl.loop(0, RPT)
    def _(r):
        z_v[r, pl.ds(0, DEGW)] = jnp.zeros((16,), jnp.float32)

    pltpu.sync_copy(z_v, acc_sh.at[pl.ds(row0, RPT)])
    pltpu.sync_copy(dst_hbm.at[pl.ds(wid * DCHUNK, DCHUNK)], dst_v)
    plsc.subcore_barrier()

    @pl.loop(0, DCHUNK)
    def _(j):
        pltpu.sync_copy(ones_v, acc_sh.at[dst_v.at[j]], add=True)

    plsc.subcore_barrier()
    pltpu.sync_copy(acc_sh.at[pl.ds(row0, RPT)],
                    out_hbm.at[cid, pl.ds(row0, RPT)])


NBUF = 8
HCHUNK = NCHUNK // 2


def _prop_body(u_hbm, src_hbm, dst_hbm, out_hbm,
               src_v, dst_v, r0, r1, r2, r3, r4, r5, r6, r7, acc_sh,
               sg0, sg1, sg2, sg3, sg4, sg5, sg6, sg7):
    cid = lax.axis_index("c")
    sid = lax.axis_index("s")
    row0 = sid * RPT
    bufs = (r0, r1, r2, r3, r4, r5, r6, r7)
    gsems = (sg0, sg1, sg2, sg3, sg4, sg5, sg6, sg7)

    @pl.loop(0, CHUNK)
    def _(r):
        @pl.loop(0, FH, step=16)
        def _(col):
            r0[r, pl.ds(col, 16)] = jnp.zeros((16,), jnp.float32)

    for k in range(4):
        pltpu.sync_copy(r0, acc_sh.at[pl.ds(row0 + k * CHUNK, CHUNK)])
    pltpu.sync_copy(r0.at[pl.ds(0, RPT - 4 * CHUNK)],
                    acc_sh.at[pl.ds(row0 + 4 * CHUNK, RPT - 4 * CHUNK)])
    plsc.subcore_barrier()

    def gcopy(j, b):
        return pltpu.make_async_copy(
            u_hbm.at[cid].at[src_v.at[j]], bufs[b], gsems[b])

    for half in range(2):
        base = sid * NCHUNK + half * HCHUNK
        pltpu.sync_copy(src_hbm.at[pl.ds(base, HCHUNK)], src_v)
        pltpu.sync_copy(dst_hbm.at[pl.ds(base, HCHUNK)], dst_v)

        for b in range(NBUF):
            gcopy(b, b).start()

        @pl.loop(0, HCHUNK // NBUF - 1)
        def _(jj):
            for b in range(NBUF):
                j = jj * NBUF + b
                gcopy(j, b).wait()
                pltpu.sync_copy(bufs[b], acc_sh.at[dst_v.at[j]], add=True)
                gcopy(j + NBUF, b).start()

        for b in range(NBUF):
            j = HCHUNK - NBUF + b
            gcopy(j, b).wait()
            pltpu.sync_copy(bufs[b], acc_sh.at[dst_v.at[j]], add=True)

    plsc.subcore_barrier()
    pltpu.sync_copy(acc_sh.at[pl.ds(row0, RPT)],
                    out_hbm.at[cid, pl.ds(row0, RPT)])


@functools.cache
def _sc_kernels():
    mesh = plsc.VectorSubcoreMesh(core_axis_name="c", subcore_axis_name="s")
    cp = pltpu.CompilerParams(use_tc_tiling_on_sc=False)
    deg = pl.kernel(
        _deg_body,
        out_type=jax.ShapeDtypeStruct((NC, NP, DEGW), jnp.float32),
        mesh=mesh,
        scratch_types=[
            pltpu.VMEM((DCHUNK, CHUNK), jnp.int32),
            pltpu.VMEM((CHUNK, DEGW), jnp.float32),
            pltpu.VMEM((RPT, DEGW), jnp.float32),
            pltpu.VMEM_SHARED((NP, DEGW), jnp.float32),
        ],
        compiler_params=cp,
    )
    prop = pl.kernel(
        _prop_body,
        out_type=jax.ShapeDtypeStruct((NC, NP, FH), jnp.float32),
        mesh=mesh,
        scratch_types=[
            pltpu.VMEM((HCHUNK, CHUNK), jnp.int32),
            pltpu.VMEM((HCHUNK, CHUNK), jnp.int32),
        ] + [pltpu.VMEM((CHUNK, FH), jnp.float32)] * NBUF + [
            pltpu.VMEM_SHARED((NP, FH), jnp.float32),
        ] + [pltpu.SemaphoreType.DMA] * NBUF,
        compiler_params=cp,
    )
    return deg, prop


def _split(a):
    return jnp.stack([a[:, :FH], a[:, FH:]])


def _unsplit(s):
    return jnp.concatenate([s[0], s[1]], axis=1)


def _matmul_body(xp_ref, w_ref, xw_ref):
    xw_ref[...] = jnp.dot(xp_ref[...], w_ref[...],
                          preferred_element_type=jnp.float32)


_matmul_call = pl.pallas_call(
    _matmul_body,
    out_shape=jax.ShapeDtypeStruct((NP, F), jnp.float32),
)


def _scale_body(dpart_ref, xw_ref, u_ref, dinv_ref):
    deg = dpart_ref[0, :, 0:1] + dpart_ref[1, :, 0:1] + 1.0
    rows = lax.broadcasted_iota(jnp.int32, (NP, 1), 0)
    dinv = jnp.where(rows < N, lax.rsqrt(deg), 0.0)
    u_ref[...] = _split(xw_ref[...] * dinv)
    dinv_ref[...] = dinv


_scale_call = pl.pallas_call(
    _scale_body,
    out_shape=(jax.ShapeDtypeStruct((NC, NP, FH), jnp.float32),
               jax.ShapeDtypeStruct((NP, 1), jnp.float32)),
)


def _layer_body(s_ref, u_ref, dinv_ref, b_ref, w_ref, u2_ref):
    dinv = dinv_ref[...]
    h = (_unsplit(s_ref[...]) + _unsplit(u_ref[...])) * dinv + b_ref[...]
    h = jnp.maximum(h, 0.0)
    xw = jnp.dot(h, w_ref[...], preferred_element_type=jnp.float32)
    u2_ref[...] = _split(xw * dinv)


_layer_call = pl.pallas_call(
    _layer_body,
    out_shape=jax.ShapeDtypeStruct((NC, NP, FH), jnp.float32),
)


SEGB = 512


def _head_body(s_ref, u_ref, dinv_ref, b_ref, batch_ref, wd_ref, bd_ref,
               out_ref, y_ref, h_s, g_ref, his_s):
    dinv = dinv_ref[...]
    h = (_unsplit(s_ref[...]) + _unsplit(u_ref[...])) * dinv + b_ref[...]
    h = jnp.maximum(h, 0.0)
    hs = h[:N]
    y_ref[...] = jnp.max(hs, axis=1, keepdims=True)
    h_s[...] = h

    gcol = lax.broadcasted_iota(jnp.int32, (NG, 1), 0)
    b_row = batch_ref[...]
    his_s[...] = jnp.sum((b_row <= gcol).astype(jnp.int32), axis=1,
                         keepdims=True)
    g_ref[...] = jnp.full((NG, F), -jnp.inf, jnp.float32)

    def per_graph(gid, lo):
        hi = his_s[pl.ds(gid, 1), :][0, 0]
        cnt = hi - lo
        nblk = (cnt + SEGB - 1) // SEGB

        def per_block(k, acc):
            start = jnp.minimum(lo + k * SEGB, NP - SEGB)
            rows = h_s[pl.ds(start, SEGB), :]
            ridx = lax.broadcasted_iota(jnp.int32, (SEGB, 1), 0) + start
            mask = (ridx >= lo) & (ridx < hi)
            seg = jnp.where(mask, rows, -jnp.inf)
            return jnp.maximum(acc, jnp.max(seg, axis=0, keepdims=True))

        row = lax.fori_loop(0, nblk, per_block,
                            jnp.full((1, F), -jnp.inf, jnp.float32))
        g_ref[pl.ds(gid, 1), :] = row
        return hi

    lax.fori_loop(0, NG, per_graph, jnp.int32(0))
    g = g_ref[...]
    g = jnp.where(jnp.isfinite(g), g, 0.0)
    logits = jnp.dot(g, wd_ref[...], preferred_element_type=jnp.float32)
    logits = logits + bd_ref[...]
    z = logits - jnp.max(logits, axis=1, keepdims=True)
    e = jnp.exp(z)
    out_ref[...] = e / jnp.sum(e, axis=1, keepdims=True)


_head_call = pl.pallas_call(
    _head_body,
    out_shape=(jax.ShapeDtypeStruct((NG, NOUT), jnp.float32),
               jax.ShapeDtypeStruct((N, 1), jnp.float32)),
    scratch_shapes=[pltpu.VMEM((NP, F), jnp.float32),
                    pltpu.VMEM((NG, F), jnp.float32),
                    pltpu.VMEM((NG, 1), jnp.int32)],
)


@jax.jit
def kernel(x, edge_index, batch, W1, b1, W2, b2, Wd, bd):
    src = edge_index[0].astype(jnp.int32)
    dst = edge_index[1].astype(jnp.int32)
    pad = jnp.full((NEP - NE,), N, jnp.int32)
    srcp = jnp.concatenate([src, pad]).reshape(E2D, CHUNK)
    dstp = jnp.concatenate([dst, pad]).reshape(E2D, CHUNK)
    xp = jnp.pad(x, ((0, NP - N), (0, 0)))

    deg_kernel, prop_kernel = _sc_kernels()
    xw1 = _matmul_call(xp, W1)
    dpart = deg_kernel(dstp)
    u1, dinv = _scale_call(dpart, xw1)
    s1 = prop_kernel(u1, srcp, dstp)
    u2 = _layer_call(s1, u1, dinv, b1.reshape(1, F), W2)
    s2 = prop_kernel(u2, srcp, dstp)
    out, y = _head_call(s2, u2, dinv, b2.reshape(1, F),
                        batch.reshape(1, N).astype(jnp.int32),
                        Wd, bd.reshape(1, NOUT))
    return (out, y)


# speedup vs baseline: 1.0840x; 1.0840x over previous
import functools

import jax
import jax.numpy as jnp
from jax import lax
from jax.experimental import pallas as pl
from jax.experimental.pallas import tpu as pltpu
from jax.experimental.pallas import tpu_sc as plsc

N = 10000
NP = 10112
NE = 320000
F = 128
NG = 64
NOUT = 10

NC, NS = 2, 16
FH = F // NC
CHUNK = 128
NCHUNK = 160
NEP = CHUNK * NCHUNK * NS
E2D = NEP // CHUNK
DCHUNK = E2D // (NC * NS)
RPT = NP // NS
DEGW = 16


def _deg_body(dst_hbm, out_hbm, dst_v, ones_v, z_v, acc_sh):
    cid = lax.axis_index("c")
    sid = lax.axis_index("s")
    wid = cid * NS + sid
    row0 = sid * RPT

    @pl.loop(0, CHUNK)
    def _(r):
        ones_v[r, pl.ds(0, DEGW)] = jnp.ones((16,), jnp.float32)

    @pl.loop(0, RPT)
    def _(r):
        z_v[r, pl.ds(0, DEGW)] = jnp.zeros((16,), jnp.float32)

    pltpu.sync_copy(z_v, acc_sh.at[pl.ds(row0, RPT)])
    pltpu.sync_copy(dst_hbm.at[pl.ds(wid * DCHUNK, DCHUNK)], dst_v)
    plsc.subcore_barrier()

    @pl.loop(0, DCHUNK)
    def _(j):
        pltpu.sync_copy(ones_v, acc_sh.at[dst_v.at[j]], add=True)

    plsc.subcore_barrier()
    pltpu.sync_copy(acc_sh.at[pl.ds(row0, RPT)],
                    out_hbm.at[cid, pl.ds(row0, RPT)])


NBUF = 8
HCHUNK = NCHUNK // 2


def _prop_body(u_hbm, src_hbm, dst_hbm, out_hbm,
               src_v, dst_v, r0, r1, r2, r3, r4, r5, r6, r7, acc_sh,
               sg0, sg1, sg2, sg3, sg4, sg5, sg6, sg7):
    cid = lax.axis_index("c")
    sid = lax.axis_index("s")
    row0 = sid * RPT
    bufs = (r0, r1, r2, r3, r4, r5, r6, r7)
    gsems = (sg0, sg1, sg2, sg3, sg4, sg5, sg6, sg7)

    @pl.loop(0, CHUNK)
    def _(r):
        @pl.loop(0, FH, step=16)
        def _(col):
            r0[r, pl.ds(col, 16)] = jnp.zeros((16,), jnp.float32)

    for k in range(4):
        pltpu.sync_copy(r0, acc_sh.at[pl.ds(row0 + k * CHUNK, CHUNK)])
    pltpu.sync_copy(r0.at[pl.ds(0, RPT - 4 * CHUNK)],
                    acc_sh.at[pl.ds(row0 + 4 * CHUNK, RPT - 4 * CHUNK)])
    plsc.subcore_barrier()

    def gcopy(j, b):
        return pltpu.make_async_copy(
            u_hbm.at[cid].at[src_v.at[j]], bufs[b], gsems[b])

    for half in range(2):
        base = sid * NCHUNK + half * HCHUNK
        pltpu.sync_copy(src_hbm.at[pl.ds(base, HCHUNK)], src_v)
        pltpu.sync_copy(dst_hbm.at[pl.ds(base, HCHUNK)], dst_v)

        for b in range(NBUF):
            gcopy(b, b).start()

        @pl.loop(0, HCHUNK // NBUF - 1)
        def _(jj):
            for b in range(NBUF):
                j = jj * NBUF + b
                gcopy(j, b).wait()
                pltpu.sync_copy(bufs[b], acc_sh.at[dst_v.at[j]], add=True)
                gcopy(j + NBUF, b).start()

        for b in range(NBUF):
            j = HCHUNK - NBUF + b
            gcopy(j, b).wait()
            pltpu.sync_copy(bufs[b], acc_sh.at[dst_v.at[j]], add=True)

    plsc.subcore_barrier()
    pltpu.sync_copy(acc_sh.at[pl.ds(row0, RPT)],
                    out_hbm.at[cid, pl.ds(row0, RPT)])


@functools.cache
def _sc_kernels():
    mesh = plsc.VectorSubcoreMesh(core_axis_name="c", subcore_axis_name="s")
    cp = pltpu.CompilerParams(use_tc_tiling_on_sc=False)
    deg = pl.kernel(
        _deg_body,
        out_type=jax.ShapeDtypeStruct((NC, NP, DEGW), jnp.float32),
        mesh=mesh,
        scratch_types=[
            pltpu.VMEM((DCHUNK, CHUNK), jnp.int32),
            pltpu.VMEM((CHUNK, DEGW), jnp.float32),
            pltpu.VMEM((RPT, DEGW), jnp.float32),
            pltpu.VMEM_SHARED((NP, DEGW), jnp.float32),
        ],
        compiler_params=cp,
    )
    prop = pl.kernel(
        _prop_body,
        out_type=jax.ShapeDtypeStruct((NC, NP, FH), jnp.float32),
        mesh=mesh,
        scratch_types=[
            pltpu.VMEM((HCHUNK, CHUNK), jnp.int32),
            pltpu.VMEM((HCHUNK, CHUNK), jnp.int32),
        ] + [pltpu.VMEM((CHUNK, FH), jnp.float32)] * NBUF + [
            pltpu.VMEM_SHARED((NP, FH), jnp.float32),
        ] + [pltpu.SemaphoreType.DMA] * NBUF,
        compiler_params=cp,
    )
    return deg, prop


def _split(a):
    return jnp.stack([a[:, :FH], a[:, FH:]])


def _unsplit(s):
    return jnp.concatenate([s[0], s[1]], axis=1)


def _scale_body(dpart_ref, xp_ref, w_ref, u_ref, dinv_ref):
    deg = dpart_ref[0, :, 0:1] + dpart_ref[1, :, 0:1] + 1.0
    rows = lax.broadcasted_iota(jnp.int32, (NP, 1), 0)
    dinv = jnp.where(rows < N, lax.rsqrt(deg), 0.0)
    xw = jnp.dot(xp_ref[...], w_ref[...], preferred_element_type=jnp.float32)
    u_ref[...] = _split(xw * dinv)
    dinv_ref[...] = dinv


_scale_call = pl.pallas_call(
    _scale_body,
    out_shape=(jax.ShapeDtypeStruct((NC, NP, FH), jnp.float32),
               jax.ShapeDtypeStruct((NP, 1), jnp.float32)),
)


def _layer_body(s_ref, u_ref, dinv_ref, b_ref, w_ref, u2_ref):
    dinv = dinv_ref[...]
    h = (_unsplit(s_ref[...]) + _unsplit(u_ref[...])) * dinv + b_ref[...]
    h = jnp.maximum(h, 0.0)
    xw = jnp.dot(h, w_ref[...], preferred_element_type=jnp.float32)
    u2_ref[...] = _split(xw * dinv)


_layer_call = pl.pallas_call(
    _layer_body,
    out_shape=jax.ShapeDtypeStruct((NC, NP, FH), jnp.float32),
)


SEGB = 512


def _head_body(s_ref, u_ref, dinv_ref, b_ref, batch_ref, wd_ref, bd_ref,
               out_ref, y_ref, h_s, g_ref, his_s):
    dinv = dinv_ref[...]
    h = (_unsplit(s_ref[...]) + _unsplit(u_ref[...])) * dinv + b_ref[...]
    h = jnp.maximum(h, 0.0)
    hs = h[:N]
    y_ref[...] = jnp.max(hs, axis=1, keepdims=True)
    h_s[...] = h

    gcol = lax.broadcasted_iota(jnp.int32, (NG, 1), 0)
    b_row = batch_ref[...]
    his_s[...] = jnp.sum((b_row <= gcol).astype(jnp.int32), axis=1,
                         keepdims=True)
    g_ref[...] = jnp.full((NG, F), -jnp.inf, jnp.float32)

    def per_graph(gid, lo):
        hi = his_s[pl.ds(gid, 1), :][0, 0]
        cnt = hi - lo
        nblk = (cnt + SEGB - 1) // SEGB

        def per_block(k, acc):
            start = jnp.minimum(lo + k * SEGB, NP - SEGB)
            rows = h_s[pl.ds(start, SEGB), :]
            ridx = lax.broadcasted_iota(jnp.int32, (SEGB, 1), 0) + start
            mask = (ridx >= lo) & (ridx < hi)
            seg = jnp.where(mask, rows, -jnp.inf)
            return jnp.maximum(acc, jnp.max(seg, axis=0, keepdims=True))

        row = lax.fori_loop(0, nblk, per_block,
                            jnp.full((1, F), -jnp.inf, jnp.float32))
        g_ref[pl.ds(gid, 1), :] = row
        return hi

    lax.fori_loop(0, NG, per_graph, jnp.int32(0))
    g = g_ref[...]
    g = jnp.where(jnp.isfinite(g), g, 0.0)
    logits = jnp.dot(g, wd_ref[...], preferred_element_type=jnp.float32)
    logits = logits + bd_ref[...]
    z = logits - jnp.max(logits, axis=1, keepdims=True)
    e = jnp.exp(z)
    out_ref[...] = e / jnp.sum(e, axis=1, keepdims=True)


_head_call = pl.pallas_call(
    _head_body,
    out_shape=(jax.ShapeDtypeStruct((NG, NOUT), jnp.float32),
               jax.ShapeDtypeStruct((N, 1), jnp.float32)),
    scratch_shapes=[pltpu.VMEM((NP, F), jnp.float32),
                    pltpu.VMEM((NG, F), jnp.float32),
                    pltpu.VMEM((NG, 1), jnp.int32)],
)


@jax.jit
def kernel(x, edge_index, batch, W1, b1, W2, b2, Wd, bd):
    src = edge_index[0].astype(jnp.int32)
    dst = edge_index[1].astype(jnp.int32)
    pad = jnp.full((NEP - NE,), N, jnp.int32)
    srcp = jnp.concatenate([src, pad]).reshape(E2D, CHUNK)
    dstp = jnp.concatenate([dst, pad]).reshape(E2D, CHUNK)
    xp = jnp.pad(x, ((0, NP - N), (0, 0)))

    deg_kernel, prop_kernel = _sc_kernels()
    dpart = deg_kernel(dstp)
    u1, dinv = _scale_call(dpart, xp, W1)
    s1 = prop_kernel(u1, srcp, dstp)
    u2 = _layer_call(s1, u1, dinv, b1.reshape(1, F), W2)
    s2 = prop_kernel(u2, srcp, dstp)
    out, y = _head_call(s2, u2, dinv, b2.reshape(1, F),
                        batch.reshape(1, N).astype(jnp.int32),
                        Wd, bd.reshape(1, NOUT))
    return (out, y)
